# baseline (speedup 1.0000x reference)
import functools

import jax
import jax.numpy as jnp
from jax import lax
from jax.experimental import pallas as pl
from jax.experimental.pallas import tpu as pltpu
from jax.experimental.pallas import tpu_sc as plsc

_B = 4
_T = 4096
_D = 1024
_NW = 32
_TW = _T // _NW
_C = 16
_H = _C // 2
_NTC = _TW // _C
_NJOB = _B * _NTC
_LANES = 16
_SL = _D // _LANES


def _emb_body(ids_hbm, tok_hbm, pos_hbm, out_hbm,
              idx_v, pos0, pos1, tk0, tk1, tk2, tk3,
              sp0, sp1, sa0, sa1, sa2, sa3, sb0, sb1, sb2, sb3,
              ss0, ss1, ss2, ss3):
    wid = lax.axis_index("s") * 2 + lax.axis_index("c")
    t0 = wid * _TW

    toks = (tk0, tk1, tk2, tk3)
    sas = (sa0, sa1, sa2, sa3)
    sbs = (sb0, sb1, sb2, sb3)
    sss = (ss0, ss1, ss2, ss3)
    poss = (pos0, pos1)
    sps = (sp0, sp1)

    for b in range(_B):
        pltpu.sync_copy(ids_hbm.at[pl.ds(b * _T + t0, _TW)],
                        idx_v.at[pl.ds(b * _TW, _TW)])

    def fire_gather(i, k):
        b = lax.rem(i, _B)
        tc = lax.div(i, _B)
        ioff = b * _TW + tc * _C
        pltpu.async_copy(tok_hbm.at[idx_v.at[pl.ds(ioff, _H)]],
                         toks[k].at[pl.ds(0, _H)], sas[k])
        pltpu.async_copy(tok_hbm.at[idx_v.at[pl.ds(ioff + _H, _H)]],
                         toks[k].at[pl.ds(_H, _H)], sbs[k])

    def fire_pos(p, pb):
        pltpu.async_copy(pos_hbm.at[pl.ds(t0 + p * _C, _C)],
                         poss[pb], sps[pb])

    fire_pos(0, 0)
    fire_gather(0, 0)
    fire_gather(1, 1)

    def add_rows(tok, posb, lo):
        def add_half(it, _):
            r = lo + lax.div(it, 2)
            base = lax.rem(it, 2) * (_SL // 2 * _LANES)
            for j in range(_SL // 2):
                sl = pl.ds(base + j * _LANES, _LANES)
                plsc.addupdate(tok.at[r, sl], posb[r, sl])
            return 0

        lax.fori_loop(0, 2 * _H, add_half, 0)

    def eight_jobs(iv, _):
        for u in range(8):
            i = iv * 8 + u
            k = u % 4
            pb = u // 4
            tok = toks[k]

            @pl.when(i >= 2)
            def _():
                pltpu.make_async_copy(toks[(k + 2) % 4],
                                      out_hbm.at[pl.ds(0, _C)],
                                      sss[(k + 2) % 4]).wait()

            @pl.when(i <= _NJOB - 3)
            def _():
                fire_gather(i + 2, (k + 2) % 4)

            if u == 0:
                pltpu.make_async_copy(pos_hbm.at[pl.ds(0, _C)],
                                      poss[0], sps[0]).wait()
                fire_pos(2 * iv + 1, 1)
            if u == 4:
                pltpu.make_async_copy(pos_hbm.at[pl.ds(0, _C)],
                                      poss[1], sps[1]).wait()

                @pl.when(2 * iv + 2 <= _NTC - 1)
                def _():
                    fire_pos(2 * iv + 2, 0)

            posb = poss[pb]
            pltpu.make_async_copy(tok_hbm.at[idx_v.at[pl.ds(0, _H)]],
                                  tok.at[pl.ds(0, _H)], sas[k]).wait()
            add_rows(tok, posb, 0)
            pltpu.make_async_copy(tok_hbm.at[idx_v.at[pl.ds(0, _H)]],
                                  tok.at[pl.ds(_H, _H)], sbs[k]).wait()
            add_rows(tok, posb, _H)

            b = lax.rem(i, _B)
            tc = lax.div(i, _B)
            row = b * _T + t0 + tc * _C
            pltpu.async_copy(tok, out_hbm.at[pl.ds(row, _C)], sss[k])
        return 0

    lax.fori_loop(0, _NJOB // 8, eight_jobs, 0)
    pltpu.make_async_copy(tk2, out_hbm.at[pl.ds(0, _C)], ss2).wait()
    pltpu.make_async_copy(tk3, out_hbm.at[pl.ds(0, _C)], ss3).wait()


@jax.jit
def _emb(ids_flat, token_emb, pos_emb):
    mesh = plsc.VectorSubcoreMesh(core_axis_name="c", subcore_axis_name="s")
    call = functools.partial(
        pl.kernel,
        mesh=mesh,
        out_type=jax.ShapeDtypeStruct((_B * _T, _D), jnp.float32),
        scratch_types=(
            [pltpu.VMEM((_B * _TW,), jnp.int32)]
            + [pltpu.VMEM((_C, _D), jnp.float32)] * 6
            + [pltpu.SemaphoreType.DMA] * 14
        ),
    )(_emb_body)
    return call(ids_flat, token_emb, pos_emb)


def kernel(input_ids, token_emb, pos_emb):
    ids_flat = input_ids.astype(jnp.int32).reshape(-1)
    out = _emb(ids_flat, token_emb, pos_emb)
    return out.reshape(_B, _T, _D)
